# baseline (speedup 1.0000x reference)
import functools

import jax
import jax.numpy as jnp
from jax import lax
from jax.experimental import pallas as pl
from jax.experimental.pallas import tpu as pltpu
from jax.experimental.pallas import tpu_sc as plsc


def _ring_copy(src_hbm, dst_hbm, base, rows, ch, scratch):
    nb = len(scratch) // 3
    bufs = scratch[:nb]
    in_sems = scratch[nb:2 * nb]
    out_sems = scratch[2 * nb:]
    nch = rows // ch
    in_copies = [None] * nb
    out_copies = [None] * nb

    for c in range(min(nb, nch)):
        in_copies[c] = pltpu.async_copy(
            src_hbm.at[pl.ds(base + c * ch, ch)], bufs[c], in_sems[c])
    for c in range(nch):
        b = c % nb
        in_copies[b].wait()
        out_copies[b] = pltpu.async_copy(
            bufs[b], dst_hbm.at[pl.ds(base + c * ch, ch)], out_sems[b])
        nxt = c + nb
        if nxt < nch:
            out_copies[b].wait()
            in_copies[b] = pltpu.async_copy(
                src_hbm.at[pl.ds(base + nxt * ch, ch)], bufs[b], in_sems[b])
    for b in range(nb):
        if out_copies[b] is not None:
            out_copies[b].wait()


def kernel(x, pe):
    S, D = pe.shape
    info = plsc.get_sparse_core_info()
    nc, ns = info.num_cores, info.num_subcores
    nw = nc * ns

    TEC_ROWS = 6144
    SCS_ROWS = S - TEC_ROWS

    CH_T = 16
    NB_T = 3
    rows_t = TEC_ROWS // nw

    CH_S = 256
    rows_s = SCS_ROWS // nc
    nch_s = rows_s // CH_S

    vmesh = plsc.VectorSubcoreMesh(core_axis_name="c", subcore_axis_name="s")
    smesh = plsc.ScalarSubcoreMesh(axis_name="c", num_cores=nc)

    def tec_fn(pe_hbm, out_hbm):
        def inner(*scratch):
            wid = lax.axis_index("s") * nc + lax.axis_index("c")
            _ring_copy(pe_hbm, out_hbm, wid * rows_t, rows_t, CH_T, scratch)
        pl.run_scoped(
            inner,
            *([pltpu.VMEM((CH_T, D), jnp.float32)] * NB_T
              + [pltpu.SemaphoreType.DMA] * (2 * NB_T)))

    NB_S = 4

    def scs_fn(pe_hbm, out_hbm, *bufs):
        def inner(*sems):
            base = TEC_ROWS + lax.axis_index("c") * rows_s
            _ring_copy(pe_hbm, out_hbm, base, rows_s, CH_S,
                       list(bufs) + list(sems))
        pl.run_scoped(inner, *([pltpu.SemaphoreType.DMA] * (2 * NB_S)))

    def tec_fn2(pe_hbm, out_hbm, *bufs):
        del bufs
        tec_fn(pe_hbm, out_hbm)

    sc_copy = pl.kernel(
        body=[tec_fn2, scs_fn],
        mesh=[vmesh, smesh],
        out_type=jax.ShapeDtypeStruct((S, D), pe.dtype),
        scratch_types=[pltpu.VMEM_SHARED((CH_S, D), jnp.float32)] * NB_S,
    )
    return sc_copy(pe)[None, :, :]


# speedup vs baseline: 1.0324x; 1.0324x over previous
import functools

import jax
import jax.numpy as jnp
from jax import lax
from jax.experimental import pallas as pl
from jax.experimental.pallas import tpu as pltpu
from jax.experimental.pallas import tpu_sc as plsc


def _ring_copy(src_hbm, dst_hbm, base, rows, ch, scratch):
    nb = len(scratch) // 3
    bufs = scratch[:nb]
    in_sems = scratch[nb:2 * nb]
    out_sems = scratch[2 * nb:]
    nch = rows // ch
    in_copies = [None] * nb
    out_copies = [None] * nb

    for c in range(min(nb, nch)):
        in_copies[c] = pltpu.async_copy(
            src_hbm.at[pl.ds(base + c * ch, ch)], bufs[c], in_sems[c])
    for c in range(nch):
        b = c % nb
        in_copies[b].wait()
        out_copies[b] = pltpu.async_copy(
            bufs[b], dst_hbm.at[pl.ds(base + c * ch, ch)], out_sems[b])
        nxt = c + nb
        if nxt < nch:
            out_copies[b].wait()
            in_copies[b] = pltpu.async_copy(
                src_hbm.at[pl.ds(base + nxt * ch, ch)], bufs[b], in_sems[b])
    for b in range(nb):
        if out_copies[b] is not None:
            out_copies[b].wait()


def kernel(x, pe):
    S, D = pe.shape
    info = plsc.get_sparse_core_info()
    nc, ns = info.num_cores, info.num_subcores
    nw = nc * ns

    TEC_ROWS = 4608
    SCS_ROWS = S - TEC_ROWS

    CH_T = 16
    NB_T = 3
    rows_t = TEC_ROWS // nw

    CH_S = 256
    rows_s = SCS_ROWS // nc
    nch_s = rows_s // CH_S

    vmesh = plsc.VectorSubcoreMesh(core_axis_name="c", subcore_axis_name="s")
    smesh = plsc.ScalarSubcoreMesh(axis_name="c", num_cores=nc)

    def tec_fn(pe_hbm, out_hbm):
        def inner(*scratch):
            wid = lax.axis_index("s") * nc + lax.axis_index("c")
            _ring_copy(pe_hbm, out_hbm, wid * rows_t, rows_t, CH_T, scratch)
        pl.run_scoped(
            inner,
            *([pltpu.VMEM((CH_T, D), jnp.float32)] * NB_T
              + [pltpu.SemaphoreType.DMA] * (2 * NB_T)))

    NB_S = 4

    def scs_fn(pe_hbm, out_hbm, *bufs):
        def inner(*sems):
            base = TEC_ROWS + lax.axis_index("c") * rows_s
            _ring_copy(pe_hbm, out_hbm, base, rows_s, CH_S,
                       list(bufs) + list(sems))
        pl.run_scoped(inner, *([pltpu.SemaphoreType.DMA] * (2 * NB_S)))

    def tec_fn2(pe_hbm, out_hbm, *bufs):
        del bufs
        tec_fn(pe_hbm, out_hbm)

    sc_copy = pl.kernel(
        body=[tec_fn2, scs_fn],
        mesh=[vmesh, smesh],
        out_type=jax.ShapeDtypeStruct((S, D), pe.dtype),
        scratch_types=[pltpu.VMEM_SHARED((CH_S, D), jnp.float32)] * NB_S,
    )
    return sc_copy(pe)[None, :, :]
